# baseline (speedup 1.0000x reference)
import functools

import jax
import jax.numpy as jnp
import numpy as np
from jax import lax
from jax.experimental import pallas as pl
from jax.experimental.pallas import tpu as pltpu
from jax.experimental.pallas import tpu_sc as plsc

N_NODES_C = 100000
MAX_DEG_C = 64
NUM_ADJ_C = 2
BATCH_C = 16384
OUT_COLS = 32

_info = plsc.get_sparse_core_info()
_NC, _NS, _L = _info.num_cores, _info.num_subcores, _info.num_lanes
_NW = _NC * _NS
_SEG = BATCH_C // 2
_HALF_ROW = N_NODES_C // 2

_PERM = np.array(
    [35, 45, 31, 63, 7, 4, 29, 44, 16, 58, 37, 19, 61, 2, 34, 5,
     30, 42, 3, 39, 56, 22, 6, 54, 18, 10, 11, 53, 32, 15, 49, 50,
     20, 43, 8, 24, 9, 40, 59, 25, 13, 52, 62, 60, 47, 33, 14, 17,
     38, 23, 0, 41, 21, 26, 57, 1, 28, 48, 36, 55, 51, 27, 12, 46],
    dtype=np.int32)


def _lane_extract(vec, lane):
    lanes = lax.iota(jnp.int32, _L)
    sel = jnp.where(lanes == jnp.full((_L,), lane, dtype=jnp.int32),
                    vec, jnp.zeros((_L,), jnp.int32))
    return jnp.max(sel)


def _sc_gather(table_t, ids, perm, ns, aid1):
    mesh = plsc.VectorSubcoreMesh(core_axis_name="c", subcore_axis_name="s")

    @functools.partial(
        pl.kernel,
        mesh=mesh,
        out_type=jax.ShapeDtypeStruct((OUT_COLS, BATCH_C), jnp.int32),
        scratch_types=[
            pltpu.VMEM((1, N_NODES_C), jnp.int32),
            pltpu.VMEM((MAX_DEG_C,), jnp.int32),
            pltpu.VMEM((2 * _L,), jnp.int32),
            pltpu.VMEM((_SEG,), jnp.int32),
            pltpu.VMEM((_SEG,), jnp.int32),
            pltpu.VMEM((_SEG,), jnp.int32),
            pltpu.SemaphoreType.DMA,
            pltpu.SemaphoreType.DMA,
            pltpu.SemaphoreType.DMA,
        ],
        compiler_params=pltpu.CompilerParams(
            needs_layout_passes=False, use_tc_tiling_on_sc=True
        ),
    )
    def k(table_hbm, ids_hbm, perm_hbm, ns_hbm, aid_hbm, out_hbm, row_v,
          perm_v, sc_v, ids_v, col_a, col_b, sem_r, sem_i, sem_o):
        w = lax.axis_index("s") * _NC + lax.axis_index("c")
        pltpu.sync_copy(perm_hbm, perm_v)
        pltpu.sync_copy(ns_hbm, sc_v.at[pl.ds(0, 1)])
        pltpu.sync_copy(aid_hbm, sc_v.at[pl.ds(_L, 1)])
        lanes = lax.iota(jnp.int32, _L)
        lane0 = lanes == jnp.zeros((_L,), jnp.int32)
        ns_vec = jnp.where(lane0, sc_v[pl.ds(0, _L)],
                           jnp.full((_L,), OUT_COLS, dtype=jnp.int32))
        start_vec = jnp.clip(ns_vec - OUT_COLS, 0, MAX_DEG_C - OUT_COLS)
        start_s = jnp.max(start_vec)
        aid_s = jnp.max(jnp.where(lane0, sc_v[pl.ds(_L, _L)],
                                  jnp.zeros((_L,), jnp.int32)))
        idx = start_s + w
        chunk = lax.shift_right_logical(idx, 4)
        lane = lax.rem(idx, _L)
        pv = [perm_v[pl.ds(i * _L, _L)] for i in range(MAX_DEG_C // _L)]
        cvec = jnp.full((_L,), chunk, dtype=jnp.int32)
        sel = jnp.where(cvec == 0, pv[0],
                        jnp.where(cvec == 1, pv[1],
                                  jnp.where(cvec == 2, pv[2], pv[3])))
        r = _lane_extract(sel, lane) + aid_s * MAX_DEG_C

        row_dma = pltpu.async_copy(table_hbm.at[pl.ds(r, 1)], row_v, sem_r)
        row_flat = row_v.at[0]
        ids0_dma = pltpu.async_copy(ids_hbm.at[pl.ds(0, _SEG)], ids_v, sem_i)
        row_dma.wait()
        ids0_dma.wait()

        def gather_seg(col_v):
            @plsc.parallel_loop(0, _SEG // _L, 1, unroll=8)
            def _(i):
                iv = ids_v[pl.ds(i * _L, _L)]
                col_v[pl.ds(i * _L, _L)] = plsc.load_gather(row_flat, [iv])

        gather_seg(col_a)
        out0_dma = pltpu.async_copy(col_a, out_hbm.at[w, pl.ds(0, _SEG)],
                                    sem_o)
        pltpu.sync_copy(ids_hbm.at[pl.ds(_SEG, _SEG)], ids_v)
        gather_seg(col_b)
        out0_dma.wait()
        pltpu.sync_copy(col_b, out_hbm.at[w, pl.ds(_SEG, _SEG)])

    return k(table_t, ids, perm, ns, aid1)


def kernel(adj_info, ids, num_samples, aid):
    table_t = adj_info.transpose(0, 2, 1).reshape(
        NUM_ADJ_C * MAX_DEG_C, N_NODES_C)
    ns = jnp.reshape(jnp.asarray(num_samples, jnp.int32), (1,))
    aid1 = jnp.reshape(jnp.asarray(aid, jnp.int32), (1,))
    out_t = _sc_gather(table_t, ids.astype(jnp.int32), jnp.asarray(_PERM),
                       ns, aid1)
    return out_t.T


# speedup vs baseline: 1.0186x; 1.0186x over previous
import functools

import jax
import jax.numpy as jnp
import numpy as np
from jax import lax
from jax.experimental import pallas as pl
from jax.experimental.pallas import tpu as pltpu
from jax.experimental.pallas import tpu_sc as plsc

N_NODES_C = 100000
MAX_DEG_C = 64
NUM_ADJ_C = 2
BATCH_C = 16384
OUT_COLS = 32

_info = plsc.get_sparse_core_info()
_NC, _NS, _L = _info.num_cores, _info.num_subcores, _info.num_lanes
_NW = _NC * _NS
_SEG = BATCH_C // 2

_PERM = np.array(
    [35, 45, 31, 63, 7, 4, 29, 44, 16, 58, 37, 19, 61, 2, 34, 5,
     30, 42, 3, 39, 56, 22, 6, 54, 18, 10, 11, 53, 32, 15, 49, 50,
     20, 43, 8, 24, 9, 40, 59, 25, 13, 52, 62, 60, 47, 33, 14, 17,
     38, 23, 0, 41, 21, 26, 57, 1, 28, 48, 36, 55, 51, 27, 12, 46],
    dtype=np.int32)


def _sc_gather(table_t, ids, row_list):
    mesh = plsc.VectorSubcoreMesh(core_axis_name="c", subcore_axis_name="s")

    @functools.partial(
        pl.kernel,
        mesh=mesh,
        out_type=jax.ShapeDtypeStruct((OUT_COLS, BATCH_C), jnp.int32),
        scratch_types=[
            pltpu.VMEM((1, N_NODES_C), jnp.int32),
            pltpu.VMEM((OUT_COLS,), jnp.int32),
            pltpu.VMEM((_SEG,), jnp.int32),
            pltpu.VMEM((_SEG,), jnp.int32),
            pltpu.VMEM((_SEG,), jnp.int32),
            pltpu.SemaphoreType.DMA,
            pltpu.SemaphoreType.DMA,
            pltpu.SemaphoreType.DMA,
        ],
        compiler_params=pltpu.CompilerParams(
            needs_layout_passes=False, use_tc_tiling_on_sc=True
        ),
    )
    def k(table_hbm, ids_hbm, rows_hbm, out_hbm, row_v, rl_v, ids_v, col_a,
          col_b, sem_r, sem_i, sem_o):
        w = lax.axis_index("s") * _NC + lax.axis_index("c")
        pltpu.sync_copy(rows_hbm, rl_v)
        lane = lax.rem(w, _L)
        half = lax.div(w, _L)
        vec = jnp.where(
            jnp.full((_L,), half, dtype=jnp.int32) == 0,
            rl_v[pl.ds(0, _L)],
            rl_v[pl.ds(_L, _L)],
        )
        lanes = lax.iota(jnp.int32, _L)
        masked = jnp.where(lanes == jnp.full((_L,), lane, dtype=jnp.int32),
                           vec, jnp.zeros((_L,), jnp.int32))
        r = jnp.max(masked)
        row_dma = pltpu.async_copy(table_hbm.at[pl.ds(r, 1)], row_v, sem_r)
        ids0_dma = pltpu.async_copy(ids_hbm.at[pl.ds(0, _SEG)], ids_v, sem_i)
        row_flat = row_v.at[0]
        row_dma.wait()
        ids0_dma.wait()

        def gather_seg(col_v):
            @plsc.parallel_loop(0, _SEG // _L, 1, unroll=8)
            def _(i):
                iv = ids_v[pl.ds(i * _L, _L)]
                col_v[pl.ds(i * _L, _L)] = plsc.load_gather(row_flat, [iv])

        gather_seg(col_a)
        out0_dma = pltpu.async_copy(col_a, out_hbm.at[w, pl.ds(0, _SEG)],
                                    sem_o)
        pltpu.sync_copy(ids_hbm.at[pl.ds(_SEG, _SEG)], ids_v)
        gather_seg(col_b)
        out0_dma.wait()
        pltpu.sync_copy(col_b, out_hbm.at[w, pl.ds(_SEG, _SEG)])

    return k(table_t, ids, row_list)


def kernel(adj_info, ids, num_samples, aid):
    table_t = adj_info.transpose(0, 2, 1).reshape(
        NUM_ADJ_C * MAX_DEG_C, N_NODES_C)
    perm = jnp.asarray(_PERM)
    start = (num_samples - OUT_COLS).astype(jnp.int32) if hasattr(
        num_samples, "astype") else jnp.int32(num_samples - OUT_COLS)
    cols = lax.dynamic_slice(perm, (start,), (OUT_COLS,))
    row_list = (cols + aid * MAX_DEG_C).astype(jnp.int32)
    out_t = _sc_gather(table_t, ids.astype(jnp.int32), row_list)
    return out_t.T
